# Initial kernel scaffold; baseline (speedup 1.0000x reference)
import jax
import jax.numpy as jnp
from jax.experimental import pallas as pl


def kernel(x, emb0, emb1):
    raise NotImplementedError("write your pallas kernel here")


# speedup vs baseline: 1.2887x; 1.2887x over previous
import functools

import jax
import jax.numpy as jnp
from jax import lax
from jax.experimental import pallas as pl
from jax.experimental.pallas import tpu as pltpu
from jax.experimental.pallas import tpu_sc as plsc

DIM0 = 100000
EMB_DIM = 128
N = 16384

NUM_CORES = 2
NUM_SUBCORES = 16
NW = NUM_CORES * NUM_SUBCORES
ROWS_PER_W = N // NW
CHUNK = 128
NCHUNK = ROWS_PER_W // CHUNK
LANES = 16


def _sc_kernel(x0_hbm, x1_hbm, emb0_hbm, emb1_hbm, out_hbm,
               idx0_v, idx1_v, buf0, buf1, sem0, sem1):
    wid = lax.axis_index("s") * NUM_CORES + lax.axis_index("c")
    base = wid * ROWS_PER_W

    pltpu.sync_copy(x0_hbm.at[wid], idx0_v)
    pltpu.sync_copy(x1_hbm.at[wid], idx1_v)

    for j in range(NCHUNK):
        cp0 = pltpu.async_copy(emb0_hbm.at[idx0_v.at[j]], buf0, sem0)
        cp1 = pltpu.async_copy(emb1_hbm.at[idx1_v.at[j]], buf1, sem1)
        cp0.wait()
        cp1.wait()

        def body(r, _):
            for c in range(EMB_DIM // LANES):
                s = pl.ds(c * LANES, LANES)
                buf0[r, s] = buf0[r, s] + buf1[r, s]
            return 0

        lax.fori_loop(0, CHUNK, body, 0)
        pltpu.sync_copy(buf0, out_hbm.at[pl.ds(base + j * CHUNK, CHUNK)])


def kernel(x, emb0, emb1):
    x = x.astype(jnp.int32)
    x0 = x[:, 0].reshape(NW, NCHUNK, CHUNK)
    x1 = x[:, 1].reshape(NW, NCHUNK, CHUNK)

    mesh = plsc.VectorSubcoreMesh(core_axis_name="c", subcore_axis_name="s")
    run = pl.kernel(
        _sc_kernel,
        mesh=mesh,
        out_type=jax.ShapeDtypeStruct((N, EMB_DIM), jnp.float32),
        scratch_types=[
            pltpu.VMEM((NCHUNK, CHUNK), jnp.int32),
            pltpu.VMEM((NCHUNK, CHUNK), jnp.int32),
            pltpu.VMEM((CHUNK, EMB_DIM), jnp.float32),
            pltpu.VMEM((CHUNK, EMB_DIM), jnp.float32),
            pltpu.SemaphoreType.DMA,
            pltpu.SemaphoreType.DMA,
        ],
    )
    return run(x0, x1, emb0, emb1)


# speedup vs baseline: 1.4447x; 1.1211x over previous
import functools

import jax
import jax.numpy as jnp
from jax import lax
from jax.experimental import pallas as pl
from jax.experimental.pallas import tpu as pltpu
from jax.experimental.pallas import tpu_sc as plsc

DIM0 = 100000
EMB_DIM = 128
N = 16384

NUM_CORES = 2
NUM_SUBCORES = 16
NW = NUM_CORES * NUM_SUBCORES
ROWS_PER_W = N // NW
CHUNK = 128
NCHUNK = ROWS_PER_W // CHUNK
LANES = 16


def _sc_kernel(x0_hbm, x1_hbm, emb0_hbm, emb1_hbm, out_hbm,
               idx0_v, idx1_v,
               g0a, g0b, g1a, g1b, oa, ob,
               gs0a, gs0b, gs1a, gs1b, ssa, ssb):
    wid = lax.axis_index("s") * NUM_CORES + lax.axis_index("c")
    base = wid * ROWS_PER_W

    g0 = (g0a, g0b)
    g1 = (g1a, g1b)
    ob_ = (oa, ob)
    gs0 = (gs0a, gs0b)
    gs1 = (gs1a, gs1b)
    ss = (ssa, ssb)

    pltpu.sync_copy(x0_hbm.at[wid], idx0_v)
    pltpu.sync_copy(x1_hbm.at[wid], idx1_v)

    def start_gather(j):
        s = j & 1
        c0 = pltpu.async_copy(emb0_hbm.at[idx0_v.at[j]], g0[s], gs0[s])
        c1 = pltpu.async_copy(emb1_hbm.at[idx1_v.at[j]], g1[s], gs1[s])
        return c0, c1

    gcp = [None] * NCHUNK
    scp = [None] * NCHUNK
    gcp[0] = start_gather(0)
    gcp[1] = start_gather(1)

    for j in range(NCHUNK):
        s = j & 1
        gcp[j][0].wait()
        gcp[j][1].wait()
        if j >= 2:
            scp[j - 2].wait()

        a, b, o = g0[s], g1[s], ob_[s]

        def body(r, _):
            for c in range(EMB_DIM // LANES):
                sl = pl.ds(c * LANES, LANES)
                o[r, sl] = a[r, sl] + b[r, sl]
            return 0

        lax.fori_loop(0, CHUNK, body, 0)

        if j + 2 < NCHUNK:
            gcp[j + 2] = start_gather(j + 2)
        scp[j] = pltpu.async_copy(
            o, out_hbm.at[pl.ds(base + j * CHUNK, CHUNK)], ss[s])

    scp[NCHUNK - 2].wait()
    scp[NCHUNK - 1].wait()


def kernel(x, emb0, emb1):
    x = x.astype(jnp.int32)
    x0 = x[:, 0].reshape(NW, NCHUNK, CHUNK)
    x1 = x[:, 1].reshape(NW, NCHUNK, CHUNK)

    mesh = plsc.VectorSubcoreMesh(core_axis_name="c", subcore_axis_name="s")
    run = pl.kernel(
        _sc_kernel,
        mesh=mesh,
        out_type=jax.ShapeDtypeStruct((N, EMB_DIM), jnp.float32),
        scratch_types=[
            pltpu.VMEM((NCHUNK, CHUNK), jnp.int32),
            pltpu.VMEM((NCHUNK, CHUNK), jnp.int32),
            pltpu.VMEM((CHUNK, EMB_DIM), jnp.float32),
            pltpu.VMEM((CHUNK, EMB_DIM), jnp.float32),
            pltpu.VMEM((CHUNK, EMB_DIM), jnp.float32),
            pltpu.VMEM((CHUNK, EMB_DIM), jnp.float32),
            pltpu.VMEM((CHUNK, EMB_DIM), jnp.float32),
            pltpu.VMEM((CHUNK, EMB_DIM), jnp.float32),
            pltpu.SemaphoreType.DMA,
            pltpu.SemaphoreType.DMA,
            pltpu.SemaphoreType.DMA,
            pltpu.SemaphoreType.DMA,
            pltpu.SemaphoreType.DMA,
            pltpu.SemaphoreType.DMA,
        ],
    )
    return run(x0, x1, emb0, emb1)


# speedup vs baseline: 1.5776x; 1.0920x over previous
import jax
import jax.numpy as jnp
from jax import lax
from jax.experimental import pallas as pl
from jax.experimental.pallas import tpu as pltpu
from jax.experimental.pallas import tpu_sc as plsc

DIM0 = 100000
EMB_DIM = 128
N = 16384

NUM_CORES = 2
NUM_SUBCORES = 16
NW = NUM_CORES * NUM_SUBCORES
ROWS_PER_W = N // NW
CHUNK = 128
NCHUNK = ROWS_PER_W // CHUNK
LANES = 16


def _sc_kernel(x0_hbm, x1_hbm, emb0_hbm, emb1_hbm, out_hbm,
               idx0_v, idx1_v,
               ba, bb, bc, bd,
               ga, gb, gc, gd, sa, sb, sc, sd):
    wid = lax.axis_index("s") * NUM_CORES + lax.axis_index("c")
    base = wid * ROWS_PER_W

    bufs = (ba, bb, bc, bd)
    gsem = (ga, gb, gc, gd)
    ssem = (sa, sb, sc, sd)

    pltpu.sync_copy(x0_hbm.at[wid], idx0_v)
    pltpu.sync_copy(x1_hbm.at[wid], idx1_v)

    cp0 = [pltpu.async_copy(emb0_hbm.at[idx0_v.at[j]], bufs[j], gsem[j])
           for j in range(NCHUNK)]
    cp1 = [None] * NCHUNK
    for j in range(NCHUNK):
        cp0[j].wait()
        cp1[j] = pltpu.async_copy(emb1_hbm.at[idx1_v.at[j]], bufs[j],
                                  gsem[j], add=True)
    st = [None] * NCHUNK
    for j in range(NCHUNK):
        cp1[j].wait()
        st[j] = pltpu.async_copy(
            bufs[j], out_hbm.at[pl.ds(base + j * CHUNK, CHUNK)], ssem[j])
    for j in range(NCHUNK):
        st[j].wait()


def kernel(x, emb0, emb1):
    x = x.astype(jnp.int32)
    x0 = x[:, 0].reshape(NW, NCHUNK, CHUNK)
    x1 = x[:, 1].reshape(NW, NCHUNK, CHUNK)

    mesh = plsc.VectorSubcoreMesh(core_axis_name="c", subcore_axis_name="s")
    run = pl.kernel(
        _sc_kernel,
        mesh=mesh,
        out_type=jax.ShapeDtypeStruct((N, EMB_DIM), jnp.float32),
        scratch_types=[
            pltpu.VMEM((NCHUNK, CHUNK), jnp.int32),
            pltpu.VMEM((NCHUNK, CHUNK), jnp.int32),
            pltpu.VMEM((CHUNK, EMB_DIM), jnp.float32),
            pltpu.VMEM((CHUNK, EMB_DIM), jnp.float32),
            pltpu.VMEM((CHUNK, EMB_DIM), jnp.float32),
            pltpu.VMEM((CHUNK, EMB_DIM), jnp.float32),
            pltpu.SemaphoreType.DMA,
            pltpu.SemaphoreType.DMA,
            pltpu.SemaphoreType.DMA,
            pltpu.SemaphoreType.DMA,
            pltpu.SemaphoreType.DMA,
            pltpu.SemaphoreType.DMA,
            pltpu.SemaphoreType.DMA,
            pltpu.SemaphoreType.DMA,
        ],
    )
    return run(x0, x1, emb0, emb1)


# speedup vs baseline: 1.5886x; 1.0070x over previous
import jax
import jax.numpy as jnp
from jax import lax
from jax.experimental import pallas as pl
from jax.experimental.pallas import tpu as pltpu
from jax.experimental.pallas import tpu_sc as plsc

DIM0 = 100000
EMB_DIM = 128
N = 16384

NUM_CORES = 2
NUM_SUBCORES = 16
NW = NUM_CORES * NUM_SUBCORES
ROWS_PER_W = N // NW
CHUNK = 64
NCHUNK = ROWS_PER_W // CHUNK


def _sc_kernel(xc_hbm, emb0_hbm, emb1_hbm, out_hbm,
               idx_v, bufv, gsem, ssem):
    wid = lax.axis_index("s") * NUM_CORES + lax.axis_index("c")
    base = wid * ROWS_PER_W

    pltpu.sync_copy(xc_hbm.at[wid], idx_v)

    cp0 = [pltpu.async_copy(emb0_hbm.at[idx_v.at[0, j]], bufv.at[j],
                            gsem.at[j])
           for j in range(NCHUNK)]
    cp1 = [None] * NCHUNK
    for j in range(NCHUNK):
        cp0[j].wait()
        cp1[j] = pltpu.async_copy(emb1_hbm.at[idx_v.at[1, j]], bufv.at[j],
                                  gsem.at[j], add=True)
    st = [None] * NCHUNK
    for j in range(NCHUNK):
        cp1[j].wait()
        st[j] = pltpu.async_copy(
            bufv.at[j], out_hbm.at[pl.ds(base + j * CHUNK, CHUNK)],
            ssem.at[j])
    for j in range(NCHUNK):
        st[j].wait()


def kernel(x, emb0, emb1):
    x = x.astype(jnp.int32)
    xc = x.reshape(NW, NCHUNK, CHUNK, 2).transpose(0, 3, 1, 2)

    mesh = plsc.VectorSubcoreMesh(core_axis_name="c", subcore_axis_name="s")
    run = pl.kernel(
        _sc_kernel,
        mesh=mesh,
        out_type=jax.ShapeDtypeStruct((N, EMB_DIM), jnp.float32),
        scratch_types=[
            pltpu.VMEM((2, NCHUNK, CHUNK), jnp.int32),
            pltpu.VMEM((NCHUNK, CHUNK, EMB_DIM), jnp.float32),
            pltpu.SemaphoreType.DMA((NCHUNK,)),
            pltpu.SemaphoreType.DMA((NCHUNK,)),
        ],
    )
    return run(xc, emb0, emb1)
